# baseline (speedup 1.0000x reference)
import functools

import jax
import jax.numpy as jnp
from jax import lax
from jax.experimental import pallas as pl
from jax.experimental.pallas import tpu as pltpu
from jax.experimental.pallas import tpu_sc as plsc

_NUM_CODES = 8192
_DIM = 64
_COMMIT = 0.25

_T = 256
_C = 2048
_N_CHUNKS = _NUM_CODES // _C


def _dist_argmin_kernel(xb_ref, xx_ref, cb_ref, cbcb_ref, idx_ref, loss_ref):
    xb = xb_ref[...]
    xx = xx_ref[...]

    acc_v = jnp.full((_T, 1), jnp.inf, jnp.float32)
    acc_i = jnp.zeros((_T, 1), jnp.int32)
    acc_sq = jnp.zeros((_T, 1), jnp.float32)

    for c in range(_N_CHUNKS):
        cb = cb_ref[pl.ds(c * _C, _C), :]
        cbcb = cbcb_ref[:, pl.ds(c * _C, _C)]
        conv = lax.dot_general(xb, cb, (((1,), (1,)), ((), ())),
                               preferred_element_type=jnp.float32)
        a = xx + cbcb
        sq = a - conv
        sqc = jnp.maximum(sq, 0.0)
        d = jnp.sqrt(sqc)
        m = jnp.min(d, axis=1, keepdims=True)
        iota = lax.broadcasted_iota(jnp.int32, (_T, _C), 1)
        i = jnp.min(jnp.where(d == m, iota, jnp.int32(2**30)),
                    axis=1, keepdims=True) + c * _C
        msq = jnp.min(sqc, axis=1, keepdims=True)
        keep = acc_v <= m
        acc_i = jnp.where(keep, acc_i, i)
        acc_sq = jnp.where(keep, acc_sq, msq)
        acc_v = jnp.minimum(acc_v, m).astype(jnp.bfloat16).astype(jnp.float32)

    idx_ref[...] = acc_i

    @pl.when(pl.program_id(0) == 0)
    def _():
        loss_ref[...] = jnp.zeros_like(loss_ref)
    loss_ref[...] = loss_ref[...] + jnp.sum(acc_sq).reshape(1, 1)


def _dist_argmin(xb, xx, codebook, cbcb):
    n = xb.shape[0]
    nb = n // _T
    return pl.pallas_call(
        _dist_argmin_kernel,
        grid=(nb,),
        in_specs=[
            pl.BlockSpec((_T, _DIM), lambda i: (i, 0)),
            pl.BlockSpec((_T, 1), lambda i: (i, 0)),
            pl.BlockSpec((_NUM_CODES, _DIM), lambda i: (0, 0)),
            pl.BlockSpec((1, _NUM_CODES), lambda i: (0, 0)),
        ],
        out_specs=[
            pl.BlockSpec((_T, 1), lambda i: (i, 0)),
            pl.BlockSpec((1, 1), lambda i: (0, 0)),
        ],
        out_shape=[
            jax.ShapeDtypeStruct((n, 1), jnp.int32),
            jax.ShapeDtypeStruct((1, 1), jnp.float32),
        ],
    )(xb, xx, codebook, cbcb)


_PAD_DIM = 128


def _sc_gather(codebook_padded, indices):
    n = indices.shape[0]
    info = plsc.get_sparse_core_info()
    nw = info.num_cores * info.num_subcores
    b_per_w = n // nw
    chunk = 96
    n_chunk = b_per_w // chunk
    mesh = plsc.VectorSubcoreMesh(core_axis_name="c", subcore_axis_name="s")

    @functools.partial(
        pl.kernel,
        out_type=jax.ShapeDtypeStruct((n, _PAD_DIM), jnp.float32),
        mesh=mesh,
        scratch_types=[
            pltpu.VMEM((n_chunk, chunk), jnp.int32),
            pltpu.VMEM((b_per_w, _PAD_DIM), jnp.float32),
            pltpu.SemaphoreType.DMA,
        ],
    )
    def gather(table_hbm, idx_hbm, out_hbm, idx_v, rows_v, sem):
        wid = lax.axis_index("s") * info.num_cores + lax.axis_index("c")
        base = wid * b_per_w
        for c in range(n_chunk):
            pltpu.sync_copy(idx_hbm.at[pl.ds(base + c * chunk, chunk)],
                            idx_v.at[c])
        copies = [pltpu.async_copy(table_hbm.at[idx_v.at[c]],
                                   rows_v.at[pl.ds(c * chunk, chunk)], sem)
                  for c in range(n_chunk)]
        for cp in copies:
            cp.wait()
        pltpu.sync_copy(rows_v, out_hbm.at[pl.ds(base, b_per_w)])

    return gather(codebook_padded, indices)


def kernel(x, codebook):
    orig_shape = x.shape
    flat = x.reshape(-1, orig_shape[-1])
    n = flat.shape[0]

    xb = (2.0 * flat).astype(jnp.bfloat16)
    xx = jnp.sum(flat * flat, axis=1, keepdims=True)
    cbcb = jnp.sum(codebook * codebook, axis=1)[None, :]
    idx2d, loss_sum = _dist_argmin(xb, xx, codebook, cbcb)
    indices_flat = idx2d.reshape(n)
    cb_padded = jnp.concatenate(
        [codebook, jnp.zeros((_NUM_CODES, _PAD_DIM - _DIM), jnp.float32)],
        axis=1)
    quantized_flat = _sc_gather(cb_padded, indices_flat)[:, :_DIM]

    mse = loss_sum[0, 0] / jnp.float32(n * _DIM)
    vq_loss = (1.0 + _COMMIT) * mse

    return (quantized_flat.reshape(orig_shape), vq_loss,
            indices_flat.reshape(orig_shape[:-1]))


# speedup vs baseline: 1.0873x; 1.0873x over previous
import functools

import jax
import jax.numpy as jnp
from jax import lax
from jax.experimental import pallas as pl
from jax.experimental.pallas import tpu as pltpu
from jax.experimental.pallas import tpu_sc as plsc

_NUM_CODES = 8192
_DIM = 64
_COMMIT = 0.25

_T = 256
_C = 2048
_N_CHUNKS = _NUM_CODES // _C


def _dist_argmin_kernel(xb_ref, xx_ref, cb_ref, cbcb_ref, idx_ref, loss_ref):
    xb = xb_ref[...]
    xx = xx_ref[...]

    acc_v = jnp.full((_T, 1), jnp.inf, jnp.float32)
    acc_i = jnp.zeros((_T, 1), jnp.int32)
    acc_sq = jnp.zeros((_T, 1), jnp.float32)
    iota = lax.broadcasted_iota(jnp.int32, (_T, _C), 1).astype(jnp.float32)

    for c in range(_N_CHUNKS):
        cb = cb_ref[pl.ds(c * _C, _C), :]
        cbcb = cbcb_ref[:, pl.ds(c * _C, _C)]
        conv = lax.dot_general(xb, cb, (((1,), (1,)), ((), ())),
                               preferred_element_type=jnp.float32)
        a = xx + cbcb
        sq = a - conv
        sqc = jnp.maximum(sq, 0.0)
        d = jnp.sqrt(sqc)
        m = jnp.min(d, axis=1, keepdims=True)
        i_f = jnp.min(jnp.where(d == m, iota, jnp.float32(2**30)),
                      axis=1, keepdims=True)
        i = i_f.astype(jnp.int32) + c * _C
        keep = acc_v <= m
        acc_i = jnp.where(keep, acc_i, i)
        acc_sq = jnp.where(keep, acc_sq, m * m)
        acc_v = jnp.minimum(acc_v, m).astype(jnp.bfloat16).astype(jnp.float32)

    idx_ref[...] = acc_i

    @pl.when(pl.program_id(0) == 0)
    def _():
        loss_ref[...] = jnp.zeros_like(loss_ref)
    loss_ref[...] = loss_ref[...] + jnp.sum(acc_sq).reshape(1, 1)


def _dist_argmin(xb, xx, codebook, cbcb):
    n = xb.shape[0]
    nb = n // _T
    return pl.pallas_call(
        _dist_argmin_kernel,
        grid=(nb,),
        in_specs=[
            pl.BlockSpec((_T, _DIM), lambda i: (i, 0)),
            pl.BlockSpec((_T, 1), lambda i: (i, 0)),
            pl.BlockSpec((_NUM_CODES, _DIM), lambda i: (0, 0)),
            pl.BlockSpec((1, _NUM_CODES), lambda i: (0, 0)),
        ],
        out_specs=[
            pl.BlockSpec((_T, 1), lambda i: (i, 0)),
            pl.BlockSpec((1, 1), lambda i: (0, 0)),
        ],
        out_shape=[
            jax.ShapeDtypeStruct((n, 1), jnp.int32),
            jax.ShapeDtypeStruct((1, 1), jnp.float32),
        ],
    )(xb, xx, codebook, cbcb)


_PAD_DIM = 128


def _sc_gather(codebook_padded, indices):
    n = indices.shape[0]
    info = plsc.get_sparse_core_info()
    nw = info.num_cores * info.num_subcores
    b_per_w = n // nw
    chunk = 96
    n_chunk = b_per_w // chunk
    mesh = plsc.VectorSubcoreMesh(core_axis_name="c", subcore_axis_name="s")

    @functools.partial(
        pl.kernel,
        out_type=jax.ShapeDtypeStruct((n, _PAD_DIM), jnp.float32),
        mesh=mesh,
        scratch_types=[
            pltpu.VMEM((n_chunk, chunk), jnp.int32),
            pltpu.VMEM((b_per_w, _PAD_DIM), jnp.float32),
            pltpu.SemaphoreType.DMA,
        ],
    )
    def gather(table_hbm, idx_hbm, out_hbm, idx_v, rows_v, sem):
        wid = lax.axis_index("s") * info.num_cores + lax.axis_index("c")
        base = wid * b_per_w
        for c in range(n_chunk):
            pltpu.sync_copy(idx_hbm.at[pl.ds(base + c * chunk, chunk)],
                            idx_v.at[c])
        copies = [pltpu.async_copy(table_hbm.at[idx_v.at[c]],
                                   rows_v.at[pl.ds(c * chunk, chunk)], sem)
                  for c in range(n_chunk)]
        for cp in copies:
            cp.wait()
        pltpu.sync_copy(rows_v, out_hbm.at[pl.ds(base, b_per_w)])

    return gather(codebook_padded, indices)


def kernel(x, codebook):
    orig_shape = x.shape
    flat = x.reshape(-1, orig_shape[-1])
    n = flat.shape[0]

    xb = (2.0 * flat).astype(jnp.bfloat16)
    xx = jnp.sum(flat * flat, axis=1, keepdims=True)
    cbcb = jnp.sum(codebook * codebook, axis=1)[None, :]
    idx2d, loss_sum = _dist_argmin(xb, xx, codebook, cbcb)
    indices_flat = idx2d.reshape(n)
    cb_padded = jnp.concatenate(
        [codebook, jnp.zeros((_NUM_CODES, _PAD_DIM - _DIM), jnp.float32)],
        axis=1)
    quantized_flat = _sc_gather(cb_padded, indices_flat)[:, :_DIM]

    mse = loss_sum[0, 0] / jnp.float32(n * _DIM)
    vq_loss = (1.0 + _COMMIT) * mse

    return (quantized_flat.reshape(orig_shape), vq_loss,
            indices_flat.reshape(orig_shape[:-1]))


# speedup vs baseline: 1.1687x; 1.0749x over previous
import functools

import jax
import jax.numpy as jnp
from jax import lax
from jax.experimental import pallas as pl
from jax.experimental.pallas import tpu as pltpu
from jax.experimental.pallas import tpu_sc as plsc

_NUM_CODES = 8192
_DIM = 64
_COMMIT = 0.25

_T = 512
_C = 2048
_N_CHUNKS = _NUM_CODES // _C


def _dist_argmin_kernel(xb_ref, xx_ref, cb_ref, cbcb_ref, idx_ref, loss_ref):
    xb = xb_ref[...]
    xx = xx_ref[...]

    acc_v = jnp.full((_T, 1), jnp.inf, jnp.float32)
    acc_i = jnp.zeros((_T, 1), jnp.int32)
    acc_sq = jnp.zeros((_T, 1), jnp.float32)
    iota = lax.broadcasted_iota(jnp.int32, (_T, _C), 1).astype(jnp.float32)

    for c in range(_N_CHUNKS):
        cb = cb_ref[pl.ds(c * _C, _C), :]
        cbcb = cbcb_ref[:, pl.ds(c * _C, _C)]
        conv = lax.dot_general(xb, cb, (((1,), (1,)), ((), ())),
                               preferred_element_type=jnp.float32)
        a = xx + cbcb
        sq = a - conv
        sqc = jnp.maximum(sq, 0.0)
        d = jnp.sqrt(sqc)
        m = jnp.min(d, axis=1, keepdims=True)
        i_f = jnp.min(jnp.where(d == m, iota, jnp.float32(2**30)),
                      axis=1, keepdims=True)
        i = i_f.astype(jnp.int32) + c * _C
        keep = acc_v <= m
        acc_i = jnp.where(keep, acc_i, i)
        acc_sq = jnp.where(keep, acc_sq, m * m)
        acc_v = jnp.minimum(acc_v, m).astype(jnp.bfloat16).astype(jnp.float32)

    idx_ref[...] = acc_i

    @pl.when(pl.program_id(0) == 0)
    def _():
        loss_ref[...] = jnp.zeros_like(loss_ref)
    loss_ref[...] = loss_ref[...] + jnp.sum(acc_sq).reshape(1, 1)


def _dist_argmin(xb, xx, codebook, cbcb):
    n = xb.shape[0]
    nb = n // _T
    return pl.pallas_call(
        _dist_argmin_kernel,
        grid=(nb,),
        in_specs=[
            pl.BlockSpec((_T, _DIM), lambda i: (i, 0)),
            pl.BlockSpec((_T, 1), lambda i: (i, 0)),
            pl.BlockSpec((_NUM_CODES, _DIM), lambda i: (0, 0)),
            pl.BlockSpec((1, _NUM_CODES), lambda i: (0, 0)),
        ],
        out_specs=[
            pl.BlockSpec((_T, 1), lambda i: (i, 0)),
            pl.BlockSpec((1, 1), lambda i: (0, 0)),
        ],
        out_shape=[
            jax.ShapeDtypeStruct((n, 1), jnp.int32),
            jax.ShapeDtypeStruct((1, 1), jnp.float32),
        ],
    )(xb, xx, codebook, cbcb)


_PAD_DIM = 128


def _sc_gather(codebook_padded, indices):
    n = indices.shape[0]
    info = plsc.get_sparse_core_info()
    nw = info.num_cores * info.num_subcores
    b_per_w = n // nw
    chunk = 96
    n_chunk = b_per_w // chunk
    mesh = plsc.VectorSubcoreMesh(core_axis_name="c", subcore_axis_name="s")

    @functools.partial(
        pl.kernel,
        out_type=jax.ShapeDtypeStruct((n, _PAD_DIM), jnp.float32),
        mesh=mesh,
        scratch_types=[
            pltpu.VMEM((n_chunk, chunk), jnp.int32),
            pltpu.VMEM((b_per_w, _PAD_DIM), jnp.float32),
            pltpu.SemaphoreType.DMA,
        ],
    )
    def gather(table_hbm, idx_hbm, out_hbm, idx_v, rows_v, sem):
        wid = lax.axis_index("s") * info.num_cores + lax.axis_index("c")
        base = wid * b_per_w
        for c in range(n_chunk):
            pltpu.sync_copy(idx_hbm.at[pl.ds(base + c * chunk, chunk)],
                            idx_v.at[c])
        copies = [pltpu.async_copy(table_hbm.at[idx_v.at[c]],
                                   rows_v.at[pl.ds(c * chunk, chunk)], sem)
                  for c in range(n_chunk)]
        for cp in copies:
            cp.wait()
        pltpu.sync_copy(rows_v, out_hbm.at[pl.ds(base, b_per_w)])

    return gather(codebook_padded, indices)


def kernel(x, codebook):
    orig_shape = x.shape
    flat = x.reshape(-1, orig_shape[-1])
    n = flat.shape[0]

    xb = (2.0 * flat).astype(jnp.bfloat16)
    xx = jnp.sum(flat * flat, axis=1, keepdims=True)
    cbcb = jnp.sum(codebook * codebook, axis=1)[None, :]
    idx2d, loss_sum = _dist_argmin(xb, xx, codebook, cbcb)
    indices_flat = idx2d.reshape(n)
    cb_padded = jnp.concatenate(
        [codebook, jnp.zeros((_NUM_CODES, _PAD_DIM - _DIM), jnp.float32)],
        axis=1)
    quantized_flat = _sc_gather(cb_padded, indices_flat)[:, :_DIM]

    mse = loss_sum[0, 0] / jnp.float32(n * _DIM)
    vq_loss = (1.0 + _COMMIT) * mse

    return (quantized_flat.reshape(orig_shape), vq_loss,
            indices_flat.reshape(orig_shape[:-1]))


# speedup vs baseline: 1.2169x; 1.0412x over previous
import functools

import jax
import jax.numpy as jnp
from jax import lax
from jax.experimental import pallas as pl
from jax.experimental.pallas import tpu as pltpu
from jax.experimental.pallas import tpu_sc as plsc

_NUM_CODES = 8192
_DIM = 64
_COMMIT = 0.25

_T = 1024
_C = 2048
_N_CHUNKS = _NUM_CODES // _C


def _dist_argmin_kernel(x_ref, xx_ref, cb_ref, cbcb_ref, idx_ref, loss_ref):
    xb = (2.0 * x_ref[...]).astype(jnp.bfloat16)
    xx = xx_ref[...]

    acc_v = jnp.full((_T, 1), jnp.inf, jnp.float32)
    acc_i = jnp.zeros((_T, 1), jnp.int32)
    acc_sq = jnp.zeros((_T, 1), jnp.float32)
    iota = lax.broadcasted_iota(jnp.int32, (_T, _C), 1).astype(jnp.float32)

    for c in range(_N_CHUNKS):
        cb = cb_ref[pl.ds(c * _C, _C), :]
        cbcb = cbcb_ref[:, pl.ds(c * _C, _C)]
        conv = lax.dot_general(xb, cb, (((1,), (1,)), ((), ())),
                               preferred_element_type=jnp.float32)
        a = xx + cbcb
        sq = a - conv
        sqc = jnp.maximum(sq, 0.0)
        d = jnp.sqrt(sqc)
        m = jnp.min(d, axis=1, keepdims=True)
        i_f = jnp.min(jnp.where(d == m, iota, jnp.float32(2**30)),
                      axis=1, keepdims=True)
        i = i_f.astype(jnp.int32) + c * _C
        keep = acc_v <= m
        acc_i = jnp.where(keep, acc_i, i)
        acc_sq = jnp.where(keep, acc_sq, m * m)
        acc_v = jnp.minimum(acc_v, m).astype(jnp.bfloat16).astype(jnp.float32)

    idx_ref[...] = acc_i

    @pl.when(pl.program_id(0) == 0)
    def _():
        loss_ref[...] = jnp.zeros_like(loss_ref)
    loss_ref[...] = loss_ref[...] + jnp.sum(acc_sq).reshape(1, 1)


def _dist_argmin(flat, xx, codebook, cbcb):
    n = flat.shape[0]
    nb = n // _T
    return pl.pallas_call(
        _dist_argmin_kernel,
        grid=(nb,),
        in_specs=[
            pl.BlockSpec((_T, _DIM), lambda i: (i, 0)),
            pl.BlockSpec((_T, 1), lambda i: (i, 0)),
            pl.BlockSpec((_NUM_CODES, _DIM), lambda i: (0, 0)),
            pl.BlockSpec((1, _NUM_CODES), lambda i: (0, 0)),
        ],
        out_specs=[
            pl.BlockSpec((_T, 1), lambda i: (i, 0)),
            pl.BlockSpec((1, 1), lambda i: (0, 0)),
        ],
        out_shape=[
            jax.ShapeDtypeStruct((n, 1), jnp.int32),
            jax.ShapeDtypeStruct((1, 1), jnp.float32),
        ],
    )(flat, xx, codebook, cbcb)


_PAD_DIM = 128


def _sc_gather(codebook_padded, indices):
    n = indices.shape[0]
    info = plsc.get_sparse_core_info()
    nw = info.num_cores * info.num_subcores
    b_per_w = n // nw
    chunk = 96
    n_chunk = b_per_w // chunk
    mesh = plsc.VectorSubcoreMesh(core_axis_name="c", subcore_axis_name="s")

    @functools.partial(
        pl.kernel,
        out_type=jax.ShapeDtypeStruct((n, _PAD_DIM), jnp.float32),
        mesh=mesh,
        scratch_types=[
            pltpu.VMEM((n_chunk, chunk), jnp.int32),
            pltpu.VMEM((b_per_w, _PAD_DIM), jnp.float32),
            pltpu.SemaphoreType.DMA,
        ],
    )
    def gather(table_hbm, idx_hbm, out_hbm, idx_v, rows_v, sem):
        wid = lax.axis_index("s") * info.num_cores + lax.axis_index("c")
        base = wid * b_per_w
        for c in range(n_chunk):
            pltpu.sync_copy(idx_hbm.at[pl.ds(base + c * chunk, chunk)],
                            idx_v.at[c])
        copies = [pltpu.async_copy(table_hbm.at[idx_v.at[c]],
                                   rows_v.at[pl.ds(c * chunk, chunk)], sem)
                  for c in range(n_chunk)]
        for cp in copies:
            cp.wait()
        pltpu.sync_copy(rows_v, out_hbm.at[pl.ds(base, b_per_w)])

    return gather(codebook_padded, indices)


def kernel(x, codebook):
    orig_shape = x.shape
    flat = x.reshape(-1, orig_shape[-1])
    n = flat.shape[0]

    xx = jnp.sum(flat * flat, axis=1, keepdims=True)
    cbcb = jnp.sum(codebook * codebook, axis=1)[None, :]
    idx2d, loss_sum = _dist_argmin(flat, xx, codebook, cbcb)
    indices_flat = idx2d.reshape(n)
    cb_padded = jnp.concatenate(
        [codebook, jnp.zeros((_NUM_CODES, _PAD_DIM - _DIM), jnp.float32)],
        axis=1)
    quantized_flat = _sc_gather(cb_padded, indices_flat)[:, :_DIM]

    mse = loss_sum[0, 0] / jnp.float32(n * _DIM)
    vq_loss = (1.0 + _COMMIT) * mse

    return (quantized_flat.reshape(orig_shape), vq_loss,
            indices_flat.reshape(orig_shape[:-1]))


# speedup vs baseline: 1.2220x; 1.0042x over previous
import functools

import jax
import jax.numpy as jnp
from jax import lax
from jax.experimental import pallas as pl
from jax.experimental.pallas import tpu as pltpu
from jax.experimental.pallas import tpu_sc as plsc

_NUM_CODES = 8192
_DIM = 64
_COMMIT = 0.25

_T = 1152
_C = 2048
_N_CHUNKS = _NUM_CODES // _C


def _dist_argmin_kernel(x_ref, xx_ref, cb_ref, cbcb_ref, idx_ref, loss_ref):
    xb = (2.0 * x_ref[...]).astype(jnp.bfloat16)
    xx = xx_ref[...]

    acc_v = jnp.full((_T, 1), jnp.inf, jnp.float32)
    acc_i = jnp.zeros((_T, 1), jnp.int32)
    acc_sq = jnp.zeros((_T, 1), jnp.float32)
    iota = lax.broadcasted_iota(jnp.int32, (_T, _C), 1).astype(jnp.float32)

    for c in range(_N_CHUNKS):
        cb = cb_ref[pl.ds(c * _C, _C), :]
        cbcb = cbcb_ref[:, pl.ds(c * _C, _C)]
        conv = lax.dot_general(xb, cb, (((1,), (1,)), ((), ())),
                               preferred_element_type=jnp.float32)
        a = xx + cbcb
        sq = a - conv
        sqc = jnp.maximum(sq, 0.0)
        d = jnp.sqrt(sqc)
        m = jnp.min(d, axis=1, keepdims=True)
        i_f = jnp.min(jnp.where(d == m, iota, jnp.float32(2**30)),
                      axis=1, keepdims=True)
        i = i_f.astype(jnp.int32) + c * _C
        keep = acc_v <= m
        acc_i = jnp.where(keep, acc_i, i)
        acc_sq = jnp.where(keep, acc_sq, m * m)
        acc_v = jnp.minimum(acc_v, m).astype(jnp.bfloat16).astype(jnp.float32)

    idx_ref[...] = acc_i

    @pl.when(pl.program_id(0) == 0)
    def _():
        loss_ref[...] = jnp.zeros_like(loss_ref)
    loss_ref[...] = loss_ref[...] + jnp.sum(acc_sq).reshape(1, 1)


def _dist_argmin(flat, xx, codebook, cbcb):
    n = flat.shape[0]
    nb = n // _T
    return pl.pallas_call(
        _dist_argmin_kernel,
        grid=(nb,),
        in_specs=[
            pl.BlockSpec((_T, _DIM), lambda i: (i, 0)),
            pl.BlockSpec((_T, 1), lambda i: (i, 0)),
            pl.BlockSpec((_NUM_CODES, _DIM), lambda i: (0, 0)),
            pl.BlockSpec((1, _NUM_CODES), lambda i: (0, 0)),
        ],
        out_specs=[
            pl.BlockSpec((_T, 1), lambda i: (i, 0)),
            pl.BlockSpec((1, 1), lambda i: (0, 0)),
        ],
        out_shape=[
            jax.ShapeDtypeStruct((n, 1), jnp.int32),
            jax.ShapeDtypeStruct((1, 1), jnp.float32),
        ],
    )(flat, xx, codebook, cbcb)


_PAD_DIM = 128


def _sc_gather(codebook_padded, indices):
    n = indices.shape[0]
    info = plsc.get_sparse_core_info()
    nw = info.num_cores * info.num_subcores
    b_per_w = n // nw
    chunk = 96
    n_chunk = b_per_w // chunk
    mesh = plsc.VectorSubcoreMesh(core_axis_name="c", subcore_axis_name="s")

    @functools.partial(
        pl.kernel,
        out_type=jax.ShapeDtypeStruct((n, _PAD_DIM), jnp.float32),
        mesh=mesh,
        scratch_types=[
            pltpu.VMEM((n_chunk, chunk), jnp.int32),
            pltpu.VMEM((b_per_w, _PAD_DIM), jnp.float32),
            pltpu.SemaphoreType.DMA,
        ],
    )
    def gather(table_hbm, idx_hbm, out_hbm, idx_v, rows_v, sem):
        wid = lax.axis_index("s") * info.num_cores + lax.axis_index("c")
        base = wid * b_per_w
        for c in range(n_chunk):
            pltpu.sync_copy(idx_hbm.at[pl.ds(base + c * chunk, chunk)],
                            idx_v.at[c])
        copies = [pltpu.async_copy(table_hbm.at[idx_v.at[c]],
                                   rows_v.at[pl.ds(c * chunk, chunk)], sem)
                  for c in range(n_chunk)]
        for cp in copies:
            cp.wait()
        pltpu.sync_copy(rows_v, out_hbm.at[pl.ds(base, b_per_w)])

    return gather(codebook_padded, indices)


def kernel(x, codebook):
    orig_shape = x.shape
    flat = x.reshape(-1, orig_shape[-1])
    n = flat.shape[0]

    xx = jnp.sum(flat * flat, axis=1, keepdims=True)
    cbcb = jnp.sum(codebook * codebook, axis=1)[None, :]
    idx2d, loss_sum = _dist_argmin(flat, xx, codebook, cbcb)
    indices_flat = idx2d.reshape(n)
    cb_padded = jnp.concatenate(
        [codebook, jnp.zeros((_NUM_CODES, _PAD_DIM - _DIM), jnp.float32)],
        axis=1)
    quantized_flat = _sc_gather(cb_padded, indices_flat)[:, :_DIM]

    mse = loss_sum[0, 0] / jnp.float32(n * _DIM)
    vq_loss = (1.0 + _COMMIT) * mse

    return (quantized_flat.reshape(orig_shape), vq_loss,
            indices_flat.reshape(orig_shape[:-1]))
